# Initial kernel scaffold; baseline (speedup 1.0000x reference)
import jax
import jax.numpy as jnp
from jax.experimental import pallas as pl


def kernel(feat_l2, feat_l3, feat_l4, feat_l5, feat_l6, boxes):
    raise NotImplementedError("write your pallas kernel here")


# speedup vs baseline: 47.0318x; 47.0318x over previous
import functools

import jax
import jax.numpy as jnp
from jax import lax
from jax.experimental import pallas as pl
from jax.experimental.pallas import tpu as pltpu
from jax.experimental.pallas import tpu_sc as plsc

B = 2
N_BOXES = 1000
C = 256
OUT = 7
CELLS = OUT * OUT
ROWS_PER_BATCH = 21845 - (21845 >> 10)
TOTAL_BOXES = B * N_BOXES

NC = 2
NS = 16
NW = NC * NS
CHUNK = -(-TOTAL_BOXES // NW)


def _splat_i32(x):
    return lax.broadcast(jnp.int32(x), (16,)) if not hasattr(x, "shape") else lax.broadcast(x, (16,))


def _sc_body(feat_hbm, boxes_hbm, out_hbm,
             boxes_v, yr0_v, yr1_v, xi0_v, xi1_v, wy0_v, wy1_v, wx0_v, wx1_v,
             idx00_v, idx01_v, idx10_v, idx11_v,
             w00_v, w01_v, w10_v, w11_v,
             r00_v, r01_v, r10_v, r11_v, out_v, sem):
    wid = lax.axis_index("s") * NC + lax.axis_index("c")
    lo = wid * CHUNK
    hi = jnp.minimum(lo + CHUNK, TOTAL_BOXES)

    pltpu.sync_copy(boxes_hbm, boxes_v)

    iota = lax.broadcasted_iota(jnp.int32, (16,), 0)
    iota_f = iota.astype(jnp.float32)

    def per_box(n, _):
        bb = lax.broadcast(n * 4, (16,))
        y1 = plsc.load_gather(boxes_v, [bb])
        x1 = plsc.load_gather(boxes_v, [bb + 1])
        y2 = plsc.load_gather(boxes_v, [bb + 2])
        x2 = plsc.load_gather(boxes_v, [bb + 3])

        h = y2 - y1
        w = x2 - x1
        area = h * w
        lvl = (jnp.int32(2)
               + jnp.where(area >= 12544.0, 1, 0)
               + jnp.where(area >= 50176.0, 1, 0)
               + jnp.where(area >= 200704.0, 1, 0)
               + jnp.where(area >= 802816.0, 1, 0))
        scale = lax.shift_left(jnp.int32(1), lvl).astype(jnp.float32)
        l2 = lvl - 2
        wl_i = lax.shift_right_logical(jnp.int32(128), l2)
        wl_f = wl_i.astype(jnp.float32)
        batch = n // N_BOXES
        base = (lax.broadcast(batch * ROWS_PER_BATCH, (16,))
                + 21845 - lax.shift_right_logical(jnp.int32(21845), 2 * l2))

        bh = h / scale
        bw = w / scale
        gy = y1 / scale + (iota_f + 0.5) * bh / 7.0
        gx = x1 / scale + (iota_f + 0.5) * bw / 7.0
        bnd = wl_f - 1.0

        y0f = jnp.maximum(gy.astype(jnp.int32).astype(jnp.float32), 0.0)
        ly = gy - y0f
        yi0 = jnp.minimum(y0f, bnd).astype(jnp.int32)
        yi1 = jnp.minimum(y0f + 1.0, bnd).astype(jnp.int32)
        x0f = jnp.maximum(gx.astype(jnp.int32).astype(jnp.float32), 0.0)
        lx = gx - x0f
        xi0 = jnp.minimum(x0f, bnd).astype(jnp.int32)
        xi1 = jnp.minimum(x0f + 1.0, bnd).astype(jnp.int32)

        yr0_v[...] = base + yi0 * wl_i
        yr1_v[...] = base + yi1 * wl_i
        xi0_v[...] = xi0
        xi1_v[...] = xi1
        wy0_v[...] = 1.0 - ly
        wy1_v[...] = ly
        wx0_v[...] = 1.0 - lx
        wx1_v[...] = lx

        for st in (0, 16, 32, 48):
            c = iota + st
            i = c // 7
            j = c - i * 7
            gy0 = plsc.load_gather(yr0_v, [i])
            gy1 = plsc.load_gather(yr1_v, [i])
            gx0 = plsc.load_gather(xi0_v, [j])
            gx1 = plsc.load_gather(xi1_v, [j])
            vwy0 = plsc.load_gather(wy0_v, [i])
            vwy1 = plsc.load_gather(wy1_v, [i])
            vwx0 = plsc.load_gather(wx0_v, [j])
            vwx1 = plsc.load_gather(wx1_v, [j])
            s = pl.ds(st, 16)
            idx00_v[s] = gy0 + gx0
            idx01_v[s] = gy0 + gx1
            idx10_v[s] = gy1 + gx0
            idx11_v[s] = gy1 + gx1
            w00_v[s] = vwy0 * vwx0
            w01_v[s] = vwy0 * vwx1
            w10_v[s] = vwy1 * vwx0
            w11_v[s] = vwy1 * vwx1

        cp0 = pltpu.async_copy(feat_hbm.at[idx00_v], r00_v, sem)
        cp1 = pltpu.async_copy(feat_hbm.at[idx01_v], r01_v, sem)
        cp2 = pltpu.async_copy(feat_hbm.at[idx10_v], r10_v, sem)
        cp3 = pltpu.async_copy(feat_hbm.at[idx11_v], r11_v, sem)
        cp0.wait()
        cp1.wait()
        cp2.wait()
        cp3.wait()

        def per_cell(cell, _):
            cs = lax.broadcast(cell, (16,))
            a00 = plsc.load_gather(w00_v, [cs])
            a01 = plsc.load_gather(w01_v, [cs])
            a10 = plsc.load_gather(w10_v, [cs])
            a11 = plsc.load_gather(w11_v, [cs])
            for t in range(C // 16):
                s = pl.ds(t * 16, 16)
                out_v[cell, s] = (a00 * r00_v[cell, s] + a01 * r01_v[cell, s]
                                  + a10 * r10_v[cell, s] + a11 * r11_v[cell, s])
            return 0

        lax.fori_loop(0, CELLS, per_cell, 0)
        pltpu.sync_copy(out_v, out_hbm.at[n])
        return 0

    lax.fori_loop(lo, hi, per_box, 0)


def kernel(feat_l2, feat_l3, feat_l4, feat_l5, feat_l6, boxes):
    feats = [feat_l2, feat_l3, feat_l4, feat_l5, feat_l6]
    flat = jnp.concatenate([f.reshape(B, -1, C) for f in feats], axis=1)
    flat = flat.reshape(B * ROWS_PER_BATCH, C)
    boxes_flat = boxes.reshape(TOTAL_BOXES * 4)

    mesh = plsc.VectorSubcoreMesh(core_axis_name="c", subcore_axis_name="s")
    run = pl.kernel(
        _sc_body,
        mesh=mesh,
        compiler_params=pltpu.CompilerParams(needs_layout_passes=False),
        out_type=jax.ShapeDtypeStruct((TOTAL_BOXES, CELLS, C), jnp.float32),
        scratch_types=[
            pltpu.VMEM((TOTAL_BOXES * 4,), jnp.float32),
            pltpu.VMEM((16,), jnp.int32),
            pltpu.VMEM((16,), jnp.int32),
            pltpu.VMEM((16,), jnp.int32),
            pltpu.VMEM((16,), jnp.int32),
            pltpu.VMEM((16,), jnp.float32),
            pltpu.VMEM((16,), jnp.float32),
            pltpu.VMEM((16,), jnp.float32),
            pltpu.VMEM((16,), jnp.float32),
            pltpu.VMEM((64,), jnp.int32),
            pltpu.VMEM((64,), jnp.int32),
            pltpu.VMEM((64,), jnp.int32),
            pltpu.VMEM((64,), jnp.int32),
            pltpu.VMEM((64,), jnp.float32),
            pltpu.VMEM((64,), jnp.float32),
            pltpu.VMEM((64,), jnp.float32),
            pltpu.VMEM((64,), jnp.float32),
            pltpu.VMEM((64, C), jnp.float32),
            pltpu.VMEM((64, C), jnp.float32),
            pltpu.VMEM((64, C), jnp.float32),
            pltpu.VMEM((64, C), jnp.float32),
            pltpu.VMEM((CELLS, C), jnp.float32),
            pltpu.SemaphoreType.DMA,
        ],
    )
    out = run(flat, boxes_flat)
    return out.reshape(B, N_BOXES, OUT, OUT, C)


# speedup vs baseline: 48.9279x; 1.0403x over previous
import functools

import jax
import jax.numpy as jnp
from jax import lax
from jax.experimental import pallas as pl
from jax.experimental.pallas import tpu as pltpu
from jax.experimental.pallas import tpu_sc as plsc

B = 2
N_BOXES = 1000
C = 256
OUT = 7
CELLS = OUT * OUT
ROWS_PER_BATCH = 21845 - (21845 >> 10)
TOTAL_BOXES = B * N_BOXES

NC = 2
NS = 16
NW = NC * NS
CHUNK = -(-TOTAL_BOXES // NW)


def _splat_i32(x):
    return lax.broadcast(jnp.int32(x), (16,)) if not hasattr(x, "shape") else lax.broadcast(x, (16,))


def _sc_body(feat_hbm, boxes_hbm, out_hbm,
             boxes_v, yr0_v, yr1_v, xi0_v, xi1_v, wy0_v, wy1_v, wx0_v, wx1_v,
             idx00_v, idx01_v, idx10_v, idx11_v,
             w00_v, w01_v, w10_v, w11_v,
             r00_v, r01_v, r10_v, r11_v, out_v, sem):
    wid = lax.axis_index("s") * NC + lax.axis_index("c")
    lo = wid * CHUNK
    hi = jnp.minimum(lo + CHUNK, TOTAL_BOXES)

    pltpu.sync_copy(boxes_hbm, boxes_v)

    iota = lax.broadcasted_iota(jnp.int32, (16,), 0)
    iota_f = iota.astype(jnp.float32)

    def per_box(n, _):
        bb = lax.broadcast(n * 4, (16,))
        y1 = plsc.load_gather(boxes_v, [bb])
        x1 = plsc.load_gather(boxes_v, [bb + 1])
        y2 = plsc.load_gather(boxes_v, [bb + 2])
        x2 = plsc.load_gather(boxes_v, [bb + 3])

        h = y2 - y1
        w = x2 - x1
        area = h * w
        lvl = (jnp.int32(2)
               + jnp.where(area >= 12544.0, 1, 0)
               + jnp.where(area >= 50176.0, 1, 0)
               + jnp.where(area >= 200704.0, 1, 0)
               + jnp.where(area >= 802816.0, 1, 0))
        scale = lax.shift_left(jnp.int32(1), lvl).astype(jnp.float32)
        l2 = lvl - 2
        wl_i = lax.shift_right_logical(jnp.int32(128), l2)
        wl_f = wl_i.astype(jnp.float32)
        batch = n // N_BOXES
        base = (lax.broadcast(batch * ROWS_PER_BATCH, (16,))
                + 21845 - lax.shift_right_logical(jnp.int32(21845), 2 * l2))

        bh = h / scale
        bw = w / scale
        gy = y1 / scale + (iota_f + 0.5) * bh / 7.0
        gx = x1 / scale + (iota_f + 0.5) * bw / 7.0
        bnd = wl_f - 1.0

        y0f = jnp.maximum(gy.astype(jnp.int32).astype(jnp.float32), 0.0)
        ly = gy - y0f
        yi0 = jnp.minimum(y0f, bnd).astype(jnp.int32)
        yi1 = jnp.minimum(y0f + 1.0, bnd).astype(jnp.int32)
        x0f = jnp.maximum(gx.astype(jnp.int32).astype(jnp.float32), 0.0)
        lx = gx - x0f
        xi0 = jnp.minimum(x0f, bnd).astype(jnp.int32)
        xi1 = jnp.minimum(x0f + 1.0, bnd).astype(jnp.int32)

        yr0_v[...] = base + yi0 * wl_i
        yr1_v[...] = base + yi1 * wl_i
        xi0_v[...] = xi0
        xi1_v[...] = xi1
        wy0_v[...] = 1.0 - ly
        wy1_v[...] = ly
        wx0_v[...] = 1.0 - lx
        wx1_v[...] = lx

        for st in (0, 16, 32, 48):
            c = iota + st
            i = c // 7
            j = c - i * 7
            gy0 = plsc.load_gather(yr0_v, [i])
            gy1 = plsc.load_gather(yr1_v, [i])
            gx0 = plsc.load_gather(xi0_v, [j])
            gx1 = plsc.load_gather(xi1_v, [j])
            vwy0 = plsc.load_gather(wy0_v, [i])
            vwy1 = plsc.load_gather(wy1_v, [i])
            vwx0 = plsc.load_gather(wx0_v, [j])
            vwx1 = plsc.load_gather(wx1_v, [j])
            s = pl.ds(st, 16)
            idx00_v[s] = gy0 + gx0
            idx01_v[s] = gy0 + gx1
            idx10_v[s] = gy1 + gx0
            idx11_v[s] = gy1 + gx1
            w00_v[s] = vwy0 * vwx0
            w01_v[s] = vwy0 * vwx1
            w10_v[s] = vwy1 * vwx0
            w11_v[s] = vwy1 * vwx1

        sl = pl.ds(0, 56)
        cp0 = pltpu.async_copy(feat_hbm.at[idx00_v.at[sl]], r00_v.at[sl], sem)
        cp1 = pltpu.async_copy(feat_hbm.at[idx01_v.at[sl]], r01_v.at[sl], sem)
        cp2 = pltpu.async_copy(feat_hbm.at[idx10_v.at[sl]], r10_v.at[sl], sem)
        cp3 = pltpu.async_copy(feat_hbm.at[idx11_v.at[sl]], r11_v.at[sl], sem)
        cp0.wait()
        cp1.wait()
        cp2.wait()
        cp3.wait()

        def per_cell(cell, _):
            cs = lax.broadcast(cell, (16,))
            a00 = plsc.load_gather(w00_v, [cs])
            a01 = plsc.load_gather(w01_v, [cs])
            a10 = plsc.load_gather(w10_v, [cs])
            a11 = plsc.load_gather(w11_v, [cs])
            for t in range(C // 16):
                s = pl.ds(t * 16, 16)
                out_v[cell, s] = (a00 * r00_v[cell, s] + a01 * r01_v[cell, s]
                                  + a10 * r10_v[cell, s] + a11 * r11_v[cell, s])
            return 0

        lax.fori_loop(0, CELLS, per_cell, 0)
        pltpu.sync_copy(out_v, out_hbm.at[n])
        return 0

    lax.fori_loop(lo, hi, per_box, 0)


def kernel(feat_l2, feat_l3, feat_l4, feat_l5, feat_l6, boxes):
    feats = [feat_l2, feat_l3, feat_l4, feat_l5, feat_l6]
    flat = jnp.concatenate([f.reshape(B, -1, C) for f in feats], axis=1)
    flat = flat.reshape(B * ROWS_PER_BATCH, C)
    boxes_flat = boxes.reshape(TOTAL_BOXES * 4)

    mesh = plsc.VectorSubcoreMesh(core_axis_name="c", subcore_axis_name="s")
    run = pl.kernel(
        _sc_body,
        mesh=mesh,
        compiler_params=pltpu.CompilerParams(needs_layout_passes=False),
        out_type=jax.ShapeDtypeStruct((TOTAL_BOXES, CELLS, C), jnp.float32),
        scratch_types=[
            pltpu.VMEM((TOTAL_BOXES * 4,), jnp.float32),
            pltpu.VMEM((16,), jnp.int32),
            pltpu.VMEM((16,), jnp.int32),
            pltpu.VMEM((16,), jnp.int32),
            pltpu.VMEM((16,), jnp.int32),
            pltpu.VMEM((16,), jnp.float32),
            pltpu.VMEM((16,), jnp.float32),
            pltpu.VMEM((16,), jnp.float32),
            pltpu.VMEM((16,), jnp.float32),
            pltpu.VMEM((64,), jnp.int32),
            pltpu.VMEM((64,), jnp.int32),
            pltpu.VMEM((64,), jnp.int32),
            pltpu.VMEM((64,), jnp.int32),
            pltpu.VMEM((64,), jnp.float32),
            pltpu.VMEM((64,), jnp.float32),
            pltpu.VMEM((64,), jnp.float32),
            pltpu.VMEM((64,), jnp.float32),
            pltpu.VMEM((64, C), jnp.float32),
            pltpu.VMEM((64, C), jnp.float32),
            pltpu.VMEM((64, C), jnp.float32),
            pltpu.VMEM((64, C), jnp.float32),
            pltpu.VMEM((CELLS, C), jnp.float32),
            pltpu.SemaphoreType.DMA,
        ],
    )
    out = run(flat, boxes_flat)
    return out.reshape(B, N_BOXES, OUT, OUT, C)
